# baseline (speedup 1.0000x reference)
import jax
import jax.numpy as jnp
from jax import lax
from jax.experimental import pallas as pl
from jax.experimental.pallas import tpu as pltpu
from jax.experimental.pallas import tpu_sc as plsc

f32 = jnp.float32
i32 = jnp.int32

NN = 50000
EE = 800000
GG = 512
EMB = 32
HID = 64
NCLS = 2

NC = 2
NS = 16
NW = NC * NS

N_PAD = 50176
HALF = N_PAD // 2
E_PAD = 803840
ER = E_PAD // 128
RPT = 392
NBR = N_PAD // 112
NBW = NBR // NW

AGG_ROWS = 25600
DEG_ROWS = 51200
CNT_ROWS = 1024

_SC_PARAMS = pltpu.CompilerParams(use_tc_tiling_on_sc=False)

_MESH = plsc.VectorSubcoreMesh(
    core_axis_name="c", subcore_axis_name="s", num_cores=NC, num_subcores=NS)


def _wid():
    return lax.axis_index("s") * NC + lax.axis_index("c")


NCHUNK = NBR // 8
NCH_IT = (NCHUNK + NW - 1) // NW


def _embhist_body(xs_hbm, xc_hbm, semb_hbm, cemb_hbm, dst2_hbm, batch2_hbm,
                  z16_hbm, o16_hbm, sh_hbm, co_hbm, degp_hbm, cntp_hbm,
                  idxs_v, idxc_v, rows_s, rows_c, dstb_v, batchb_v,
                  zeros_v, ones_v, deg_sh, cnt_sh, sem):
    c = lax.axis_index("c")
    s = lax.axis_index("s")
    w = _wid()
    pltpu.sync_copy(z16_hbm, zeros_v)
    pltpu.sync_copy(o16_hbm, ones_v)

    def zb(k, carry):
        pltpu.sync_copy(zeros_v, deg_sh.at[pl.ds(s * 3200 + k * 128, 128)])
        return carry
    lax.fori_loop(0, 25, zb, 0)
    pltpu.sync_copy(zeros_v.at[pl.ds(0, 64)], cnt_sh.at[pl.ds(s * 64, 64)])
    plsc.subcore_barrier()

    ech = ER // 8

    def eb(t, carry):
        ch = t * NW + w

        @pl.when(ch < ech)
        def _():
            pltpu.sync_copy(dst2_hbm.at[pl.ds(ch * 8, 8)], dstb_v)
            for j in range(8):
                pltpu.sync_copy(ones_v, deg_sh.at[dstb_v.at[j]], add=True)
        return carry
    lax.fori_loop(0, (ech + NW - 1) // NW, eb, 0)

    for t in range(NCH_IT):
        ch = t * NW + w

        @pl.when(ch < NCHUNK)
        def _():
            r0 = ch * 8
            pltpu.sync_copy(batch2_hbm.at[pl.ds(r0, 8)], batchb_v)
            for j in range(8):
                pltpu.sync_copy(ones_v.at[pl.ds(0, 112)],
                                cnt_sh.at[batchb_v.at[j]], add=True)

    for t in range(NCH_IT):
        ch = t * NW + w

        @pl.when(ch < NCHUNK)
        def _():
            r0 = ch * 8
            pltpu.sync_copy(xs_hbm.at[pl.ds(r0, 8)], idxs_v)
            pltpu.sync_copy(xc_hbm.at[pl.ds(r0, 8)], idxc_v)
            for j in range(8):
                nb = (r0 + j) * 112
                pltpu.async_copy(semb_hbm.at[idxs_v.at[j]], rows_s, sem).wait()
                pltpu.sync_copy(rows_s, sh_hbm.at[pl.ds(nb, 112)])
                pltpu.async_copy(cemb_hbm.at[idxc_v.at[j]], rows_c, sem).wait()
                pltpu.sync_copy(rows_c, co_hbm.at[pl.ds(nb, 112)])
    plsc.subcore_barrier()

    def wb(k, carry):
        r = s * 3200 + k * 128
        pltpu.sync_copy(deg_sh.at[pl.ds(r, 128)],
                        degp_hbm.at[pl.ds(c * DEG_ROWS + r, 128)])
        return carry
    lax.fori_loop(0, 25, wb, 0)
    pltpu.sync_copy(cnt_sh.at[pl.ds(s * 64, 64)],
                    cntp_hbm.at[pl.ds(c * CNT_ROWS + s * 64, 64)])


def _embhist_call(xs2, xc2, semb, cemb, dst2, batch2, z16, o16):
    return pl.kernel(
        _embhist_body,
        out_type=(jax.ShapeDtypeStruct((N_PAD, EMB), f32),
                  jax.ShapeDtypeStruct((N_PAD, EMB), f32),
                  jax.ShapeDtypeStruct((NC * DEG_ROWS, 16), f32),
                  jax.ShapeDtypeStruct((NC * CNT_ROWS, 16), f32)),
        mesh=_MESH,
        compiler_params=_SC_PARAMS,
        scratch_types=(
            pltpu.VMEM((8, 112), i32),
            pltpu.VMEM((8, 112), i32),
            pltpu.VMEM((112, EMB), f32),
            pltpu.VMEM((112, EMB), f32),
            pltpu.VMEM((8, 128), i32),
            pltpu.VMEM((8, 112), i32),
            pltpu.VMEM((128, 16), f32),
            pltpu.VMEM((128, 16), f32),
            pltpu.VMEM_SHARED((DEG_ROWS, 16), f32),
            pltpu.VMEM_SHARED((CNT_ROWS, 16), f32),
            pltpu.SemaphoreType.DMA,
        ),
    )(xs2, xc2, semb, cemb, dst2, batch2, z16, o16)


def _agg_body(h_hbm, e2_hbm, z64_hbm, out_hbm,
              idxb_v, dl_v, rows_v, agg_sh, gs0, gs1, gs2, ss0, ss1, ss2,
              is0, is1, is2):
    c = lax.axis_index("c")
    s = lax.axis_index("s")
    base = c * HALF
    gs = (gs0, gs1, gs2)
    ss = (ss0, ss1, ss2)
    isx = (is0, is1, is2)
    pltpu.sync_copy(z64_hbm, rows_v.at[0])

    def zb(k, carry):
        pltpu.sync_copy(rows_v.at[0], agg_sh.at[pl.ds(s * 1600 + k * 128, 128)])
        return carry
    lax.fori_loop(0, 12, zb, 0)
    pltpu.sync_copy(rows_v.at[0, pl.ds(0, 64)],
                    agg_sh.at[pl.ds(s * 1600 + 1536, 64)])
    plsc.subcore_barrier()

    er0 = s * RPT

    for r in range(3):
        pltpu.sync_copy(e2_hbm.at[er0 + r], idxb_v.at[r])
    for r in range(2):
        pltpu.async_copy(h_hbm.at[idxb_v.at[r, 0]], rows_v.at[r], gs[r])

    def slot(t, j):
        jg = (j + 2) % 3
        for i in range(8):
            v = idxb_v[j, 1, pl.ds(i * 16, 16)]
            u = v - base
            m = (u >= 0) & (u < HALF)
            dl_v[j, pl.ds(i * 16, 16)] = jnp.where(m, u, HALF + (v & 511))

        @pl.when(t >= 1)
        def _():
            pltpu.make_async_copy(
                rows_v.at[jg], agg_sh.at[dl_v.at[jg]], ss[jg]).wait()

        @pl.when((t >= 1) & (t < RPT - 2))
        def _():
            pltpu.make_async_copy(e2_hbm.at[er0], idxb_v.at[jg],
                                  isx[jg]).wait()

        @pl.when(t < RPT - 2)
        def _():
            pltpu.async_copy(h_hbm.at[idxb_v.at[jg, 0]], rows_v.at[jg],
                             gs[jg])
        pltpu.make_async_copy(h_hbm.at[idxb_v.at[j, 0]], rows_v.at[j],
                              gs[j]).wait()

        @pl.when(t < RPT - 3)
        def _():
            pltpu.async_copy(e2_hbm.at[er0 + t + 3], idxb_v.at[j], isx[j])
        pltpu.async_copy(rows_v.at[j], agg_sh.at[dl_v.at[j]], ss[j],
                         add=True)

    def eb(k, carry):
        for j in range(3):
            slot(3 * k + j, j)
        return carry
    lax.fori_loop(0, (RPT - 2) // 3, eb, 0)
    for t in range(RPT - 2, RPT):
        slot(jnp.int32(t), t % 3)
    pltpu.make_async_copy(rows_v.at[(RPT - 1) % 3],
                          agg_sh.at[dl_v.at[(RPT - 1) % 3]],
                          ss[(RPT - 1) % 3]).wait()
    plsc.subcore_barrier()

    o0 = c * HALF + s * 1568

    def wb(k, carry):
        pltpu.sync_copy(agg_sh.at[pl.ds(s * 1568 + k * 128, 128)],
                        out_hbm.at[pl.ds(o0 + k * 128, 128)])
        return carry
    lax.fori_loop(0, 12, wb, 0)
    pltpu.sync_copy(agg_sh.at[pl.ds(s * 1568 + 1536, 32)],
                    out_hbm.at[pl.ds(o0 + 1536, 32)])


def _agg_call(h, e2, z64):
    return pl.kernel(
        _agg_body,
        out_type=jax.ShapeDtypeStruct((N_PAD, HID), f32),
        mesh=_MESH,
        compiler_params=_SC_PARAMS,
        scratch_types=(
            pltpu.VMEM((3, 2, 128), i32),
            pltpu.VMEM((3, 128), i32),
            pltpu.VMEM((3, 128, HID), f32),
            pltpu.VMEM_SHARED((AGG_ROWS, HID), f32),
            pltpu.SemaphoreType.DMA,
            pltpu.SemaphoreType.DMA,
            pltpu.SemaphoreType.DMA,
            pltpu.SemaphoreType.DMA,
            pltpu.SemaphoreType.DMA,
            pltpu.SemaphoreType.DMA,
            pltpu.SemaphoreType.DMA,
            pltpu.SemaphoreType.DMA,
            pltpu.SemaphoreType.DMA,
        ),
    )(h, e2, z64)


def _pool_body(h_hbm, batch2_hbm, z64_hbm, out_hbm,
               hb_v, batchb_v, zeros_v, pool_sh):
    c = lax.axis_index("c")
    s = lax.axis_index("s")
    w = _wid()
    pltpu.sync_copy(z64_hbm, zeros_v)
    pltpu.sync_copy(zeros_v.at[pl.ds(0, 64)], pool_sh.at[pl.ds(s * 64, 64)])
    plsc.subcore_barrier()
    for t in range(NCH_IT):
        ch = t * NW + w

        @pl.when(ch < NCHUNK)
        def _():
            pltpu.sync_copy(batch2_hbm.at[pl.ds(ch * 8, 8)], batchb_v)
            for j in range(8):
                node0 = (ch * 8 + j) * 112
                pltpu.sync_copy(h_hbm.at[pl.ds(node0, 112)], hb_v)
                pltpu.sync_copy(hb_v, pool_sh.at[batchb_v.at[j]], add=True)
    plsc.subcore_barrier()
    pltpu.sync_copy(pool_sh.at[pl.ds(s * 64, 64)],
                    out_hbm.at[pl.ds(c * CNT_ROWS + s * 64, 64)])


def _pool_call(h2, batch2, z64):
    return pl.kernel(
        _pool_body,
        out_type=jax.ShapeDtypeStruct((NC * CNT_ROWS, HID), f32),
        mesh=_MESH,
        compiler_params=_SC_PARAMS,
        scratch_types=(
            pltpu.VMEM((112, HID), f32),
            pltpu.VMEM((8, 112), i32),
            pltpu.VMEM((128, HID), f32),
            pltpu.VMEM_SHARED((CNT_ROWS, HID), f32),
        ),
    )(h2, batch2, z64)


_RB = 1568
_NRB = N_PAD // _RB

def _dot_t(a, b):
    return lax.dot_general(a, b, (((1,), (1,)), ((), ())),
                           preferred_element_type=f32)


def _inproj_body(sh_ref, co_ref, a1_ref, a2_ref, b_ref, d0_ref, d1_ref,
                 o_ref, inv_ref):
    t = _dot_t(sh_ref[...], a1_ref[...]) + _dot_t(co_ref[...], a2_ref[...])
    o_ref[...] = jnp.maximum(t + b_ref[...], 0.0)
    d = d0_ref[0, :, 0] + d1_ref[0, :, 0]
    inv_ref[...] = (1.0 / jnp.maximum(d, 1.0)).reshape(_RB, 1)


def _inproj_call(sh, co, a1, a2, b, degp3):
    return pl.pallas_call(
        _inproj_body,
        grid=(_NRB,),
        in_specs=[
            pl.BlockSpec((_RB, EMB), lambda i: (i, 0)),
            pl.BlockSpec((_RB, EMB), lambda i: (i, 0)),
            pl.BlockSpec((HID, EMB), lambda i: (0, 0)),
            pl.BlockSpec((HID, EMB), lambda i: (0, 0)),
            pl.BlockSpec((1, HID), lambda i: (0, 0)),
            pl.BlockSpec((1, _RB, 16), lambda i: (0, i, 0)),
            pl.BlockSpec((1, _RB, 16), lambda i: (1, i, 0)),
        ],
        out_specs=[
            pl.BlockSpec((_RB, HID), lambda i: (i, 0)),
            pl.BlockSpec((_RB, 1), lambda i: (i, 0)),
        ],
        out_shape=[
            jax.ShapeDtypeStruct((N_PAD, HID), f32),
            jax.ShapeDtypeStruct((N_PAD, 1), f32),
        ],
    )(sh, co, a1, a2, b, degp3, degp3)


def _sage_body(sums_ref, h_ref, inv_ref, wl_ref, wr_ref, bl_ref, o_ref):
    agg = sums_ref[...] * inv_ref[...]
    t = _dot_t(agg, wl_ref[...]) + bl_ref[...] + _dot_t(h_ref[...], wr_ref[...])
    o_ref[...] = jnp.maximum(t, 0.0)


def _sage_call(sums, h, invd, wl, wr, bl):
    return pl.pallas_call(
        _sage_body,
        grid=(_NRB,),
        in_specs=[
            pl.BlockSpec((_RB, HID), lambda i: (i, 0)),
            pl.BlockSpec((_RB, HID), lambda i: (i, 0)),
            pl.BlockSpec((_RB, 1), lambda i: (i, 0)),
            pl.BlockSpec((HID, HID), lambda i: (0, 0)),
            pl.BlockSpec((HID, HID), lambda i: (0, 0)),
            pl.BlockSpec((1, HID), lambda i: (0, 0)),
        ],
        out_specs=pl.BlockSpec((_RB, HID), lambda i: (i, 0)),
        out_shape=jax.ShapeDtypeStruct((N_PAD, HID), f32),
    )(sums, h, invd, wl, wr, bl)


def _out_body(pool_ref, cnt_ref, wo_ref, bo_ref, o_ref):
    p = pool_ref[0:GG] + pool_ref[CNT_ROWS:CNT_ROWS + GG]
    cc = cnt_ref[0:GG, 0] + cnt_ref[CNT_ROWS:CNT_ROWS + GG, 0]
    pooled = p * (1.0 / jnp.maximum(cc, 1.0)).reshape(GG, 1)
    o_ref[...] = _dot_t(pooled, wo_ref[...]) + bo_ref[...]


def _out_call(poolp, cntp, wo, bo):
    return pl.pallas_call(
        _out_body,
        out_shape=jax.ShapeDtypeStruct((GG, 128), f32),
    )(poolp, cntp, wo, bo)


def kernel(x, edge_index, batch, shape_emb, color_emb, W_in, b_in,
           Wl1, Wr1, bl1, Wl2, Wr2, bl2, W_out, b_out):
    src2 = jnp.pad(edge_index[0].astype(i32), (0, E_PAD - EE)).reshape(ER, 128)
    dst2 = jnp.pad(edge_index[1].astype(i32), (0, E_PAD - EE),
                   constant_values=N_PAD).reshape(ER, 128)
    e2 = jnp.stack([src2, dst2], axis=1)
    xs2 = jnp.pad(x[:, 0].astype(i32), (0, N_PAD - NN)).reshape(NBR, 112)
    xc2 = jnp.pad(x[:, 1].astype(i32), (0, N_PAD - NN)).reshape(NBR, 112)
    b2 = jnp.pad(batch.astype(i32), (0, N_PAD - NN),
                 constant_values=GG).reshape(NBR, 112)
    z64 = jnp.zeros((128, HID), f32)
    z16 = jnp.zeros((128, 16), f32)
    o16 = jnp.ones((128, 16), f32)

    sh, co, degp, cntp = _embhist_call(xs2, xc2, shape_emb, color_emb,
                                       dst2, b2, z16, o16)
    h0, invd = _inproj_call(sh, co, W_in[:, :EMB], W_in[:, EMB:],
                            b_in.reshape(1, HID),
                            degp.reshape(NC, DEG_ROWS, 16))

    s1 = _agg_call(h0, e2, z64)
    h1 = _sage_call(s1, h0, invd, Wl1, Wr1, bl1.reshape(1, HID))
    s2 = _agg_call(h1, e2, z64)
    h2 = _sage_call(s2, h1, invd, Wl2, Wr2, bl2.reshape(1, HID))

    poolp = _pool_call(h2, b2, z64)
    wo = jnp.zeros((128, HID), f32).at[:NCLS].set(W_out)
    bo = jnp.zeros((1, 128), f32).at[0, :NCLS].set(b_out)
    outp = _out_call(poolp, cntp, wo, bo)
    return outp[:, :NCLS]


# speedup vs baseline: 1.0144x; 1.0144x over previous
import jax
import jax.numpy as jnp
from jax import lax
from jax.experimental import pallas as pl
from jax.experimental.pallas import tpu as pltpu
from jax.experimental.pallas import tpu_sc as plsc

f32 = jnp.float32
i32 = jnp.int32

NN = 50000
EE = 800000
GG = 512
EMB = 32
HID = 64
NCLS = 2

NC = 2
NS = 16
NW = NC * NS

N_PAD = 50176
HALF = N_PAD // 2
E_PAD = 803840
ER = E_PAD // 128
RPT = 392
NBR = N_PAD // 112
NBW = NBR // NW

AGG_ROWS = 25600
DEG_ROWS = 51200
CNT_ROWS = 1024

_SC_PARAMS = pltpu.CompilerParams(use_tc_tiling_on_sc=False)

_MESH = plsc.VectorSubcoreMesh(
    core_axis_name="c", subcore_axis_name="s", num_cores=NC, num_subcores=NS)


def _wid():
    return lax.axis_index("s") * NC + lax.axis_index("c")


NCHUNK = NBR // 8
NCH_IT = (NCHUNK + NW - 1) // NW


def _emb_body(xs_hbm, xc_hbm, semb_hbm, cemb_hbm, sh_hbm, co_hbm,
              idxs_v, idxc_v, rows_s, rows_c, sem):
    w = _wid()
    for t in range(NCH_IT):
        ch = t * NW + w

        @pl.when(ch < NCHUNK)
        def _():
            r0 = ch * 8
            pltpu.sync_copy(xs_hbm.at[pl.ds(r0, 8)], idxs_v)
            pltpu.sync_copy(xc_hbm.at[pl.ds(r0, 8)], idxc_v)
            for j in range(8):
                nb = (r0 + j) * 112
                pltpu.async_copy(semb_hbm.at[idxs_v.at[j]], rows_s, sem).wait()
                pltpu.sync_copy(rows_s, sh_hbm.at[pl.ds(nb, 112)])
                pltpu.async_copy(cemb_hbm.at[idxc_v.at[j]], rows_c, sem).wait()
                pltpu.sync_copy(rows_c, co_hbm.at[pl.ds(nb, 112)])


def _emb_call(xs2, xc2, semb, cemb):
    return pl.kernel(
        _emb_body,
        out_type=(jax.ShapeDtypeStruct((N_PAD, EMB), f32),
                  jax.ShapeDtypeStruct((N_PAD, EMB), f32)),
        mesh=_MESH,
        compiler_params=_SC_PARAMS,
        scratch_types=(
            pltpu.VMEM((8, 112), i32),
            pltpu.VMEM((8, 112), i32),
            pltpu.VMEM((112, EMB), f32),
            pltpu.VMEM((112, EMB), f32),
            pltpu.SemaphoreType.DMA,
        ),
    )(xs2, xc2, semb, cemb)


def _hist_body(dst2_hbm, batch2_hbm, z16_hbm, o16_hbm, degp_hbm, cntp_hbm,
               dstb_v, batchb_v, zeros_v, ones_v, deg_sh, cnt_sh):
    c = lax.axis_index("c")
    s = lax.axis_index("s")
    w = _wid()
    pltpu.sync_copy(z16_hbm, zeros_v)
    pltpu.sync_copy(o16_hbm, ones_v)

    def zb(k, carry):
        pltpu.sync_copy(zeros_v, deg_sh.at[pl.ds(s * 3200 + k * 128, 128)])
        return carry
    lax.fori_loop(0, 25, zb, 0)
    pltpu.sync_copy(zeros_v.at[pl.ds(0, 64)], cnt_sh.at[pl.ds(s * 64, 64)])
    plsc.subcore_barrier()

    ech = ER // 8

    def eb(t, carry):
        ch = t * NW + w

        @pl.when(ch < ech)
        def _():
            pltpu.sync_copy(dst2_hbm.at[pl.ds(ch * 8, 8)], dstb_v)
            for j in range(8):
                pltpu.sync_copy(ones_v, deg_sh.at[dstb_v.at[j]], add=True)
        return carry
    lax.fori_loop(0, (ech + NW - 1) // NW, eb, 0)

    for t in range(NCH_IT):
        ch = t * NW + w

        @pl.when(ch < NCHUNK)
        def _():
            r0 = ch * 8
            pltpu.sync_copy(batch2_hbm.at[pl.ds(r0, 8)], batchb_v)
            for j in range(8):
                pltpu.sync_copy(ones_v.at[pl.ds(0, 112)],
                                cnt_sh.at[batchb_v.at[j]], add=True)
    plsc.subcore_barrier()

    def wb(k, carry):
        r = s * 3200 + k * 128
        pltpu.sync_copy(deg_sh.at[pl.ds(r, 128)],
                        degp_hbm.at[pl.ds(c * DEG_ROWS + r, 128)])
        return carry
    lax.fori_loop(0, 25, wb, 0)
    pltpu.sync_copy(cnt_sh.at[pl.ds(s * 64, 64)],
                    cntp_hbm.at[pl.ds(c * CNT_ROWS + s * 64, 64)])


def _hist_call(dst2, batch2, z16, o16):
    return pl.kernel(
        _hist_body,
        out_type=(jax.ShapeDtypeStruct((NC * DEG_ROWS, 16), f32),
                  jax.ShapeDtypeStruct((NC * CNT_ROWS, 16), f32)),
        mesh=_MESH,
        compiler_params=_SC_PARAMS,
        scratch_types=(
            pltpu.VMEM((8, 128), i32),
            pltpu.VMEM((8, 112), i32),
            pltpu.VMEM((128, 16), f32),
            pltpu.VMEM((128, 16), f32),
            pltpu.VMEM_SHARED((DEG_ROWS, 16), f32),
            pltpu.VMEM_SHARED((CNT_ROWS, 16), f32),
        ),
    )(dst2, batch2, z16, o16)


def _agg_body(h_hbm, e2_hbm, z64_hbm, out_hbm,
              idxb_v, dl_v, rows_v, agg_sh, gs0, gs1, gs2, ss0, ss1, ss2,
              is0, is1, is2):
    c = lax.axis_index("c")
    s = lax.axis_index("s")
    base = c * HALF
    gs = (gs0, gs1, gs2)
    ss = (ss0, ss1, ss2)
    isx = (is0, is1, is2)
    pltpu.sync_copy(z64_hbm, rows_v.at[0])

    def zb(k, carry):
        pltpu.sync_copy(rows_v.at[0], agg_sh.at[pl.ds(s * 1600 + k * 128, 128)])
        return carry
    lax.fori_loop(0, 12, zb, 0)
    pltpu.sync_copy(rows_v.at[0, pl.ds(0, 64)],
                    agg_sh.at[pl.ds(s * 1600 + 1536, 64)])
    plsc.subcore_barrier()

    er0 = s * RPT

    for r in range(3):
        pltpu.sync_copy(e2_hbm.at[er0 + r], idxb_v.at[r])
    for r in range(2):
        pltpu.async_copy(h_hbm.at[idxb_v.at[r, 0]], rows_v.at[r], gs[r])

    def slot(t, j):
        jg = (j + 2) % 3
        for i in range(8):
            v = idxb_v[j, 1, pl.ds(i * 16, 16)]
            u = v - base
            m = (u >= 0) & (u < HALF)
            dl_v[j, pl.ds(i * 16, 16)] = jnp.where(m, u, HALF + (v & 511))

        @pl.when(t >= 1)
        def _():
            pltpu.make_async_copy(
                rows_v.at[jg], agg_sh.at[dl_v.at[jg]], ss[jg]).wait()

        @pl.when((t >= 1) & (t < RPT - 2))
        def _():
            pltpu.make_async_copy(e2_hbm.at[er0], idxb_v.at[jg],
                                  isx[jg]).wait()

        @pl.when(t < RPT - 2)
        def _():
            pltpu.async_copy(h_hbm.at[idxb_v.at[jg, 0]], rows_v.at[jg],
                             gs[jg])
        pltpu.make_async_copy(h_hbm.at[idxb_v.at[j, 0]], rows_v.at[j],
                              gs[j]).wait()

        @pl.when(t < RPT - 3)
        def _():
            pltpu.async_copy(e2_hbm.at[er0 + t + 3], idxb_v.at[j], isx[j])
        pltpu.async_copy(rows_v.at[j], agg_sh.at[dl_v.at[j]], ss[j],
                         add=True)

    def eb(k, carry):
        for j in range(3):
            slot(3 * k + j, j)
        return carry
    lax.fori_loop(0, (RPT - 2) // 3, eb, 0)
    for t in range(RPT - 2, RPT):
        slot(jnp.int32(t), t % 3)
    pltpu.make_async_copy(rows_v.at[(RPT - 1) % 3],
                          agg_sh.at[dl_v.at[(RPT - 1) % 3]],
                          ss[(RPT - 1) % 3]).wait()
    plsc.subcore_barrier()

    o0 = c * HALF + s * 1568

    def wb(k, carry):
        pltpu.sync_copy(agg_sh.at[pl.ds(s * 1568 + k * 128, 128)],
                        out_hbm.at[pl.ds(o0 + k * 128, 128)])
        return carry
    lax.fori_loop(0, 12, wb, 0)
    pltpu.sync_copy(agg_sh.at[pl.ds(s * 1568 + 1536, 32)],
                    out_hbm.at[pl.ds(o0 + 1536, 32)])


def _agg_call(h, e2, z64):
    return pl.kernel(
        _agg_body,
        out_type=jax.ShapeDtypeStruct((N_PAD, HID), f32),
        mesh=_MESH,
        compiler_params=_SC_PARAMS,
        scratch_types=(
            pltpu.VMEM((3, 2, 128), i32),
            pltpu.VMEM((3, 128), i32),
            pltpu.VMEM((3, 128, HID), f32),
            pltpu.VMEM_SHARED((AGG_ROWS, HID), f32),
            pltpu.SemaphoreType.DMA,
            pltpu.SemaphoreType.DMA,
            pltpu.SemaphoreType.DMA,
            pltpu.SemaphoreType.DMA,
            pltpu.SemaphoreType.DMA,
            pltpu.SemaphoreType.DMA,
            pltpu.SemaphoreType.DMA,
            pltpu.SemaphoreType.DMA,
            pltpu.SemaphoreType.DMA,
        ),
    )(h, e2, z64)


def _pool_body(h_hbm, batch2_hbm, z64_hbm, out_hbm,
               hb_v, batchb_v, zeros_v, pool_sh):
    c = lax.axis_index("c")
    s = lax.axis_index("s")
    w = _wid()
    pltpu.sync_copy(z64_hbm, zeros_v)
    pltpu.sync_copy(zeros_v.at[pl.ds(0, 64)], pool_sh.at[pl.ds(s * 64, 64)])
    plsc.subcore_barrier()
    for t in range(NCH_IT):
        ch = t * NW + w

        @pl.when(ch < NCHUNK)
        def _():
            pltpu.sync_copy(batch2_hbm.at[pl.ds(ch * 8, 8)], batchb_v)
            for j in range(8):
                node0 = (ch * 8 + j) * 112
                pltpu.sync_copy(h_hbm.at[pl.ds(node0, 112)], hb_v)
                pltpu.sync_copy(hb_v, pool_sh.at[batchb_v.at[j]], add=True)
    plsc.subcore_barrier()
    pltpu.sync_copy(pool_sh.at[pl.ds(s * 64, 64)],
                    out_hbm.at[pl.ds(c * CNT_ROWS + s * 64, 64)])


def _pool_call(h2, batch2, z64):
    return pl.kernel(
        _pool_body,
        out_type=jax.ShapeDtypeStruct((NC * CNT_ROWS, HID), f32),
        mesh=_MESH,
        compiler_params=_SC_PARAMS,
        scratch_types=(
            pltpu.VMEM((112, HID), f32),
            pltpu.VMEM((8, 112), i32),
            pltpu.VMEM((128, HID), f32),
            pltpu.VMEM_SHARED((CNT_ROWS, HID), f32),
        ),
    )(h2, batch2, z64)


_RB = 1568
_NRB = N_PAD // _RB

def _dot_t(a, b):
    return lax.dot_general(a, b, (((1,), (1,)), ((), ())),
                           preferred_element_type=f32)


def _inproj_body(sh_ref, co_ref, a1_ref, a2_ref, b_ref, d0_ref, d1_ref,
                 o_ref, inv_ref):
    t = _dot_t(sh_ref[...], a1_ref[...]) + _dot_t(co_ref[...], a2_ref[...])
    o_ref[...] = jnp.maximum(t + b_ref[...], 0.0)
    d = d0_ref[0, :, 0] + d1_ref[0, :, 0]
    inv_ref[...] = (1.0 / jnp.maximum(d, 1.0)).reshape(_RB, 1)


def _inproj_call(sh, co, a1, a2, b, degp3):
    return pl.pallas_call(
        _inproj_body,
        grid=(_NRB,),
        in_specs=[
            pl.BlockSpec((_RB, EMB), lambda i: (i, 0)),
            pl.BlockSpec((_RB, EMB), lambda i: (i, 0)),
            pl.BlockSpec((HID, EMB), lambda i: (0, 0)),
            pl.BlockSpec((HID, EMB), lambda i: (0, 0)),
            pl.BlockSpec((1, HID), lambda i: (0, 0)),
            pl.BlockSpec((1, _RB, 16), lambda i: (0, i, 0)),
            pl.BlockSpec((1, _RB, 16), lambda i: (1, i, 0)),
        ],
        out_specs=[
            pl.BlockSpec((_RB, HID), lambda i: (i, 0)),
            pl.BlockSpec((_RB, 1), lambda i: (i, 0)),
        ],
        out_shape=[
            jax.ShapeDtypeStruct((N_PAD, HID), f32),
            jax.ShapeDtypeStruct((N_PAD, 1), f32),
        ],
    )(sh, co, a1, a2, b, degp3, degp3)


def _sage_body(sums_ref, h_ref, inv_ref, wl_ref, wr_ref, bl_ref, o_ref):
    agg = sums_ref[...] * inv_ref[...]
    t = _dot_t(agg, wl_ref[...]) + bl_ref[...] + _dot_t(h_ref[...], wr_ref[...])
    o_ref[...] = jnp.maximum(t, 0.0)


def _sage_call(sums, h, invd, wl, wr, bl):
    return pl.pallas_call(
        _sage_body,
        grid=(_NRB,),
        in_specs=[
            pl.BlockSpec((_RB, HID), lambda i: (i, 0)),
            pl.BlockSpec((_RB, HID), lambda i: (i, 0)),
            pl.BlockSpec((_RB, 1), lambda i: (i, 0)),
            pl.BlockSpec((HID, HID), lambda i: (0, 0)),
            pl.BlockSpec((HID, HID), lambda i: (0, 0)),
            pl.BlockSpec((1, HID), lambda i: (0, 0)),
        ],
        out_specs=pl.BlockSpec((_RB, HID), lambda i: (i, 0)),
        out_shape=jax.ShapeDtypeStruct((N_PAD, HID), f32),
    )(sums, h, invd, wl, wr, bl)


def _out_body(pool_ref, cnt_ref, wo_ref, bo_ref, o_ref):
    p = pool_ref[0:GG] + pool_ref[CNT_ROWS:CNT_ROWS + GG]
    cc = cnt_ref[0:GG, 0] + cnt_ref[CNT_ROWS:CNT_ROWS + GG, 0]
    pooled = p * (1.0 / jnp.maximum(cc, 1.0)).reshape(GG, 1)
    o_ref[...] = _dot_t(pooled, wo_ref[...]) + bo_ref[...]


def _out_call(poolp, cntp, wo, bo):
    return pl.pallas_call(
        _out_body,
        out_shape=jax.ShapeDtypeStruct((GG, 128), f32),
    )(poolp, cntp, wo, bo)


def kernel(x, edge_index, batch, shape_emb, color_emb, W_in, b_in,
           Wl1, Wr1, bl1, Wl2, Wr2, bl2, W_out, b_out):
    src2 = jnp.pad(edge_index[0].astype(i32), (0, E_PAD - EE)).reshape(ER, 128)
    dst2 = jnp.pad(edge_index[1].astype(i32), (0, E_PAD - EE),
                   constant_values=N_PAD).reshape(ER, 128)
    e2 = jnp.stack([src2, dst2], axis=1)
    xs2 = jnp.pad(x[:, 0].astype(i32), (0, N_PAD - NN)).reshape(NBR, 112)
    xc2 = jnp.pad(x[:, 1].astype(i32), (0, N_PAD - NN)).reshape(NBR, 112)
    b2 = jnp.pad(batch.astype(i32), (0, N_PAD - NN),
                 constant_values=GG).reshape(NBR, 112)
    z64 = jnp.zeros((128, HID), f32)
    z16 = jnp.zeros((128, 16), f32)
    o16 = jnp.ones((128, 16), f32)

    sh, co = _emb_call(xs2, xc2, shape_emb, color_emb)
    degp, cntp = _hist_call(dst2, b2, z16, o16)
    h0, invd = _inproj_call(sh, co, W_in[:, :EMB], W_in[:, EMB:],
                            b_in.reshape(1, HID),
                            degp.reshape(NC, DEG_ROWS, 16))

    s1 = _agg_call(h0, e2, z64)
    h1 = _sage_call(s1, h0, invd, Wl1, Wr1, bl1.reshape(1, HID))
    s2 = _agg_call(h1, e2, z64)
    h2 = _sage_call(s2, h1, invd, Wl2, Wr2, bl2.reshape(1, HID))

    poolp = _pool_call(h2, b2, z64)
    wo = jnp.zeros((128, HID), f32).at[:NCLS].set(W_out)
    bo = jnp.zeros((1, 128), f32).at[0, :NCLS].set(b_out)
    outp = _out_call(poolp, cntp, wo, bo)
    return outp[:, :NCLS]


# speedup vs baseline: 1.1085x; 1.0927x over previous
import jax
import jax.numpy as jnp
from jax import lax
from jax.experimental import pallas as pl
from jax.experimental.pallas import tpu as pltpu
from jax.experimental.pallas import tpu_sc as plsc

f32 = jnp.float32
i32 = jnp.int32

NN = 50000
EE = 800000
GG = 512
EMB = 32
HID = 64
NCLS = 2

NC = 2
NS = 16
NW = NC * NS

N_PAD = 50176
HALF = N_PAD // 2
E_PAD = 803840
ER = E_PAD // 128
RPT = 392
NBR = N_PAD // 112
NBW = NBR // NW

AGG_ROWS = 25600
DEG_ROWS = 51200
CNT_ROWS = 1024

_SC_PARAMS = pltpu.CompilerParams(use_tc_tiling_on_sc=False)

_MESH = plsc.VectorSubcoreMesh(
    core_axis_name="c", subcore_axis_name="s", num_cores=NC, num_subcores=NS)


def _wid():
    return lax.axis_index("s") * NC + lax.axis_index("c")


NCHUNK = NBR // 8
NCH_IT = (NCHUNK + NW - 1) // NW


def _emb_body(xs_hbm, xc_hbm, semb_hbm, cemb_hbm, sh_hbm, co_hbm,
              idxs_v, idxc_v, rows_s, rows_c, sem):
    w = _wid()
    for t in range(NCH_IT):
        ch = t * NW + w

        @pl.when(ch < NCHUNK)
        def _():
            r0 = ch * 8
            pltpu.sync_copy(xs_hbm.at[pl.ds(r0, 8)], idxs_v)
            pltpu.sync_copy(xc_hbm.at[pl.ds(r0, 8)], idxc_v)
            for j in range(8):
                nb = (r0 + j) * 112
                pltpu.async_copy(semb_hbm.at[idxs_v.at[j]], rows_s, sem).wait()
                pltpu.sync_copy(rows_s, sh_hbm.at[pl.ds(nb, 112)])
                pltpu.async_copy(cemb_hbm.at[idxc_v.at[j]], rows_c, sem).wait()
                pltpu.sync_copy(rows_c, co_hbm.at[pl.ds(nb, 112)])


def _emb_call(xs2, xc2, semb, cemb):
    return pl.kernel(
        _emb_body,
        out_type=(jax.ShapeDtypeStruct((N_PAD, EMB), f32),
                  jax.ShapeDtypeStruct((N_PAD, EMB), f32)),
        mesh=_MESH,
        compiler_params=_SC_PARAMS,
        scratch_types=(
            pltpu.VMEM((8, 112), i32),
            pltpu.VMEM((8, 112), i32),
            pltpu.VMEM((112, EMB), f32),
            pltpu.VMEM((112, EMB), f32),
            pltpu.SemaphoreType.DMA,
        ),
    )(xs2, xc2, semb, cemb)


def _hist_body(dst2_hbm, batch2_hbm, z16_hbm, o16_hbm, degp_hbm, cntp_hbm,
               dstb_v, batchb_v, zeros_v, ones_v, deg_sh, cnt_sh):
    c = lax.axis_index("c")
    s = lax.axis_index("s")
    w = _wid()
    pltpu.sync_copy(z16_hbm, zeros_v)
    pltpu.sync_copy(o16_hbm, ones_v)

    def zb(k, carry):
        pltpu.sync_copy(zeros_v, deg_sh.at[pl.ds(s * 3200 + k * 128, 128)])
        return carry
    lax.fori_loop(0, 25, zb, 0)
    pltpu.sync_copy(zeros_v.at[pl.ds(0, 64)], cnt_sh.at[pl.ds(s * 64, 64)])
    plsc.subcore_barrier()

    ech = ER // 8

    def eb(t, carry):
        ch = t * NW + w

        @pl.when(ch < ech)
        def _():
            pltpu.sync_copy(dst2_hbm.at[pl.ds(ch * 8, 8)], dstb_v)
            for j in range(8):
                pltpu.sync_copy(ones_v, deg_sh.at[dstb_v.at[j]], add=True)
        return carry
    lax.fori_loop(0, (ech + NW - 1) // NW, eb, 0)

    for t in range(NCH_IT):
        ch = t * NW + w

        @pl.when(ch < NCHUNK)
        def _():
            r0 = ch * 8
            pltpu.sync_copy(batch2_hbm.at[pl.ds(r0, 8)], batchb_v)
            for j in range(8):
                pltpu.sync_copy(ones_v.at[pl.ds(0, 112)],
                                cnt_sh.at[batchb_v.at[j]], add=True)
    plsc.subcore_barrier()

    def wb(k, carry):
        r = s * 3200 + k * 128
        pltpu.sync_copy(deg_sh.at[pl.ds(r, 128)],
                        degp_hbm.at[pl.ds(c * DEG_ROWS + r, 128)])
        return carry
    lax.fori_loop(0, 25, wb, 0)
    pltpu.sync_copy(cnt_sh.at[pl.ds(s * 64, 64)],
                    cntp_hbm.at[pl.ds(c * CNT_ROWS + s * 64, 64)])


def _hist_call(dst2, batch2, z16, o16):
    return pl.kernel(
        _hist_body,
        out_type=(jax.ShapeDtypeStruct((NC * DEG_ROWS, 16), f32),
                  jax.ShapeDtypeStruct((NC * CNT_ROWS, 16), f32)),
        mesh=_MESH,
        compiler_params=_SC_PARAMS,
        scratch_types=(
            pltpu.VMEM((8, 128), i32),
            pltpu.VMEM((8, 112), i32),
            pltpu.VMEM((128, 16), f32),
            pltpu.VMEM((128, 16), f32),
            pltpu.VMEM_SHARED((DEG_ROWS, 16), f32),
            pltpu.VMEM_SHARED((CNT_ROWS, 16), f32),
        ),
    )(dst2, batch2, z16, o16)


def _agg_body(h_hbm, e2_hbm, z64_hbm, out_hbm,
              idxb_v, dl_v, rows_v, agg_sh, gs0, gs1, gs2, ss0, ss1, ss2,
              is0, is1, is2):
    c = lax.axis_index("c")
    s = lax.axis_index("s")
    base = c * HALF
    gs = (gs0, gs1, gs2)
    ss = (ss0, ss1, ss2)
    isx = (is0, is1, is2)
    pltpu.sync_copy(z64_hbm, rows_v.at[0])

    def zb(k, carry):
        pltpu.sync_copy(rows_v.at[0], agg_sh.at[pl.ds(s * 1600 + k * 128, 128)])
        return carry
    lax.fori_loop(0, 12, zb, 0)
    pltpu.sync_copy(rows_v.at[0, pl.ds(0, 64)],
                    agg_sh.at[pl.ds(s * 1600 + 1536, 64)])
    plsc.subcore_barrier()

    er0 = s * RPT

    for r in range(3):
        pltpu.sync_copy(e2_hbm.at[er0 + r], idxb_v.at[r])
    for r in range(2):
        pltpu.async_copy(h_hbm.at[idxb_v.at[r, 0]], rows_v.at[r], gs[r])

    def slot(t, j):
        jg = (j + 2) % 3
        for i in range(8):
            v = idxb_v[j, 1, pl.ds(i * 16, 16)]
            u = v - base
            m = (u >= 0) & (u < HALF)
            dl_v[j, pl.ds(i * 16, 16)] = jnp.where(m, u, HALF + (v & 511))

        @pl.when(t >= 1)
        def _():
            pltpu.make_async_copy(
                rows_v.at[jg], agg_sh.at[dl_v.at[jg]], ss[jg]).wait()

        @pl.when((t >= 1) & (t < RPT - 2))
        def _():
            pltpu.make_async_copy(e2_hbm.at[er0], idxb_v.at[jg],
                                  isx[jg]).wait()

        @pl.when(t < RPT - 2)
        def _():
            pltpu.async_copy(h_hbm.at[idxb_v.at[jg, 0]], rows_v.at[jg],
                             gs[jg])
        pltpu.make_async_copy(h_hbm.at[idxb_v.at[j, 0]], rows_v.at[j],
                              gs[j]).wait()

        @pl.when(t < RPT - 3)
        def _():
            pltpu.async_copy(e2_hbm.at[er0 + t + 3], idxb_v.at[j], isx[j])
        pltpu.async_copy(rows_v.at[j], agg_sh.at[dl_v.at[j]], ss[j],
                         add=True)

    def eb(k, carry):
        for j in range(3):
            slot(3 * k + j, j)
        return carry
    lax.fori_loop(0, (RPT - 2) // 3, eb, 0)
    for t in range(RPT - 2, RPT):
        slot(jnp.int32(t), t % 3)
    pltpu.make_async_copy(rows_v.at[(RPT - 1) % 3],
                          agg_sh.at[dl_v.at[(RPT - 1) % 3]],
                          ss[(RPT - 1) % 3]).wait()
    plsc.subcore_barrier()

    o0 = c * HALF + s * 1568

    def wb(k, carry):
        pltpu.sync_copy(agg_sh.at[pl.ds(s * 1568 + k * 128, 128)],
                        out_hbm.at[pl.ds(o0 + k * 128, 128)])
        return carry
    lax.fori_loop(0, 12, wb, 0)
    pltpu.sync_copy(agg_sh.at[pl.ds(s * 1568 + 1536, 32)],
                    out_hbm.at[pl.ds(o0 + 1536, 32)])


def _agg_call(h, e2, z64):
    return pl.kernel(
        _agg_body,
        out_type=jax.ShapeDtypeStruct((N_PAD, HID), f32),
        mesh=_MESH,
        compiler_params=_SC_PARAMS,
        scratch_types=(
            pltpu.VMEM((3, 2, 128), i32),
            pltpu.VMEM((3, 128), i32),
            pltpu.VMEM((3, 128, HID), f32),
            pltpu.VMEM_SHARED((AGG_ROWS, HID), f32),
            pltpu.SemaphoreType.DMA,
            pltpu.SemaphoreType.DMA,
            pltpu.SemaphoreType.DMA,
            pltpu.SemaphoreType.DMA,
            pltpu.SemaphoreType.DMA,
            pltpu.SemaphoreType.DMA,
            pltpu.SemaphoreType.DMA,
            pltpu.SemaphoreType.DMA,
            pltpu.SemaphoreType.DMA,
        ),
    )(h, e2, z64)


def _pool_body(h_hbm, batch2_hbm, z64_hbm, out_hbm,
               hb_v, batchb_v, zeros_v, pool_sh):
    c = lax.axis_index("c")
    s = lax.axis_index("s")
    w = _wid()
    pltpu.sync_copy(z64_hbm, zeros_v)
    pltpu.sync_copy(zeros_v.at[pl.ds(0, 64)], pool_sh.at[pl.ds(s * 64, 64)])
    plsc.subcore_barrier()
    for t in range(NCH_IT):
        ch = t * NW + w

        @pl.when(ch < NCHUNK)
        def _():
            pltpu.sync_copy(batch2_hbm.at[pl.ds(ch * 8, 8)], batchb_v)
            for j in range(8):
                node0 = (ch * 8 + j) * 112
                pltpu.sync_copy(h_hbm.at[pl.ds(node0, 112)], hb_v)
                pltpu.sync_copy(hb_v, pool_sh.at[batchb_v.at[j]], add=True)
    plsc.subcore_barrier()
    pltpu.sync_copy(pool_sh.at[pl.ds(s * 64, 64)],
                    out_hbm.at[pl.ds(c * CNT_ROWS + s * 64, 64)])


def _pool_call(h2, batch2, z64):
    return pl.kernel(
        _pool_body,
        out_type=jax.ShapeDtypeStruct((NC * CNT_ROWS, HID), f32),
        mesh=_MESH,
        compiler_params=_SC_PARAMS,
        scratch_types=(
            pltpu.VMEM((112, HID), f32),
            pltpu.VMEM((8, 112), i32),
            pltpu.VMEM((128, HID), f32),
            pltpu.VMEM_SHARED((CNT_ROWS, HID), f32),
        ),
    )(h2, batch2, z64)


_RB = 1568
_NRB = N_PAD // _RB

def _dot_t(a, b):
    return lax.dot_general(a, b, (((1,), (1,)), ((), ())),
                           preferred_element_type=f32)


def _inproj_body(sh_ref, co_ref, a1_ref, a2_ref, b_ref, o_ref):
    t = _dot_t(sh_ref[...], a1_ref[...]) + _dot_t(co_ref[...], a2_ref[...])
    o_ref[...] = jnp.maximum(t + b_ref[...], 0.0)


def _inproj_call(sh, co, a1, a2, b):
    return pl.pallas_call(
        _inproj_body,
        grid=(_NRB,),
        in_specs=[
            pl.BlockSpec((_RB, EMB), lambda i: (i, 0)),
            pl.BlockSpec((_RB, EMB), lambda i: (i, 0)),
            pl.BlockSpec((HID, EMB), lambda i: (0, 0)),
            pl.BlockSpec((HID, EMB), lambda i: (0, 0)),
            pl.BlockSpec((1, HID), lambda i: (0, 0)),
        ],
        out_specs=pl.BlockSpec((_RB, HID), lambda i: (i, 0)),
        out_shape=jax.ShapeDtypeStruct((N_PAD, HID), f32),
    )(sh, co, a1, a2, b)


def _invdeg_body(d0_ref, d1_ref, o_ref):
    d = d0_ref[0, :, 0] + d1_ref[0, :, 0]
    o_ref[...] = (1.0 / jnp.maximum(d, 1.0)).reshape(_RB, 1)


def _invdeg_call(degp3):
    return pl.pallas_call(
        _invdeg_body,
        grid=(_NRB,),
        in_specs=[
            pl.BlockSpec((1, _RB, 16), lambda i: (0, i, 0)),
            pl.BlockSpec((1, _RB, 16), lambda i: (1, i, 0)),
        ],
        out_specs=pl.BlockSpec((_RB, 1), lambda i: (i, 0)),
        out_shape=jax.ShapeDtypeStruct((N_PAD, 1), f32),
    )(degp3, degp3)


def _sage_body(sums_ref, h_ref, inv_ref, wl_ref, wr_ref, bl_ref, o_ref):
    agg = sums_ref[...] * inv_ref[...]
    t = _dot_t(agg, wl_ref[...]) + bl_ref[...] + _dot_t(h_ref[...], wr_ref[...])
    o_ref[...] = jnp.maximum(t, 0.0)


def _sage_call(sums, h, invd, wl, wr, bl):
    return pl.pallas_call(
        _sage_body,
        grid=(_NRB,),
        in_specs=[
            pl.BlockSpec((_RB, HID), lambda i: (i, 0)),
            pl.BlockSpec((_RB, HID), lambda i: (i, 0)),
            pl.BlockSpec((_RB, 1), lambda i: (i, 0)),
            pl.BlockSpec((HID, HID), lambda i: (0, 0)),
            pl.BlockSpec((HID, HID), lambda i: (0, 0)),
            pl.BlockSpec((1, HID), lambda i: (0, 0)),
        ],
        out_specs=pl.BlockSpec((_RB, HID), lambda i: (i, 0)),
        out_shape=jax.ShapeDtypeStruct((N_PAD, HID), f32),
    )(sums, h, invd, wl, wr, bl)


def _out_body(pool_ref, cnt_ref, wo_ref, bo_ref, o_ref):
    p = pool_ref[0:GG] + pool_ref[CNT_ROWS:CNT_ROWS + GG]
    cc = cnt_ref[0:GG, 0] + cnt_ref[CNT_ROWS:CNT_ROWS + GG, 0]
    pooled = p * (1.0 / jnp.maximum(cc, 1.0)).reshape(GG, 1)
    o_ref[...] = _dot_t(pooled, wo_ref[...]) + bo_ref[...]


def _out_call(poolp, cntp, wo, bo):
    return pl.pallas_call(
        _out_body,
        out_shape=jax.ShapeDtypeStruct((GG, 128), f32),
    )(poolp, cntp, wo, bo)


def kernel(x, edge_index, batch, shape_emb, color_emb, W_in, b_in,
           Wl1, Wr1, bl1, Wl2, Wr2, bl2, W_out, b_out):
    src2 = jnp.pad(edge_index[0].astype(i32), (0, E_PAD - EE)).reshape(ER, 128)
    dst2 = jnp.pad(edge_index[1].astype(i32), (0, E_PAD - EE),
                   constant_values=N_PAD).reshape(ER, 128)
    e2 = jnp.stack([src2, dst2], axis=1)
    xs2 = jnp.pad(x[:, 0].astype(i32), (0, N_PAD - NN)).reshape(NBR, 112)
    xc2 = jnp.pad(x[:, 1].astype(i32), (0, N_PAD - NN)).reshape(NBR, 112)
    b2 = jnp.pad(batch.astype(i32), (0, N_PAD - NN),
                 constant_values=GG).reshape(NBR, 112)
    z64 = jnp.zeros((128, HID), f32)
    z16 = jnp.zeros((128, 16), f32)
    o16 = jnp.ones((128, 16), f32)

    sh, co = _emb_call(xs2, xc2, shape_emb, color_emb)
    h0 = _inproj_call(sh, co, W_in[:, :EMB], W_in[:, EMB:],
                      b_in.reshape(1, HID))
    degp, cntp = _hist_call(dst2, b2, z16, o16)
    invd = _invdeg_call(degp.reshape(NC, DEG_ROWS, 16))

    s1 = _agg_call(h0, e2, z64)
    h1 = _sage_call(s1, h0, invd, Wl1, Wr1, bl1.reshape(1, HID))
    s2 = _agg_call(h1, e2, z64)
    h2 = _sage_call(s2, h1, invd, Wl2, Wr2, bl2.reshape(1, HID))

    poolp = _pool_call(h2, b2, z64)
    wo = jnp.zeros((128, HID), f32).at[:NCLS].set(W_out)
    bo = jnp.zeros((1, 128), f32).at[0, :NCLS].set(b_out)
    outp = _out_call(poolp, cntp, wo, bo)
    return outp[:, :NCLS]
